# baseline (speedup 1.0000x reference)
import functools

import jax
import jax.numpy as jnp
from jax import lax
from jax.experimental import pallas as pl
from jax.experimental.pallas import tpu as pltpu
from jax.experimental.pallas import tpu_sc as plsc

_N = 5000
_NP = 5120
_B = 128
_NB = _NP // _B
_T = 0.5
_MAXD = 100

_NW = 32
_RPW = _NP // _NW
_CH = 80
_NCH = _NP // _CH
_W = 16


def _sc_gather_body(boxes_hbm, s_hbm, order_hbm, out4_hbm, outs_hbm,
                    idx_v, rows_v, svals_v, sem):
    wid = lax.axis_index("s") * 2 + lax.axis_index("c")
    base = wid * (_RPW // _CH)
    pltpu.sync_copy(order_hbm.at[pl.ds(base, 2)], idx_v)
    c0 = pltpu.async_copy(boxes_hbm.at[idx_v.at[0]], rows_v.at[0], sem)
    c1 = pltpu.async_copy(boxes_hbm.at[idx_v.at[1]], rows_v.at[1], sem)
    c2 = pltpu.async_copy(s_hbm.at[idx_v.at[0]], svals_v.at[0], sem)
    c3 = pltpu.async_copy(s_hbm.at[idx_v.at[1]], svals_v.at[1], sem)
    c0.wait()
    c1.wait()
    c2.wait()
    c3.wait()
    pltpu.sync_copy(rows_v, out4_hbm.at[pl.ds(base, 2)])
    pltpu.sync_copy(svals_v, outs_hbm.at[pl.ds(base, 2)])


def _sc_gather(boxes_p, s_p, order):
    fn = pl.kernel(
        _sc_gather_body,
        out_type=(jax.ShapeDtypeStruct((_NCH, _CH, 4), jnp.float32),
                  jax.ShapeDtypeStruct((_NCH, _CH), jnp.float32)),
        mesh=plsc.VectorSubcoreMesh(core_axis_name="c", subcore_axis_name="s"),
        scratch_types=[
            pltpu.VMEM((2, _CH), jnp.int32),
            pltpu.VMEM((2, _CH, 4), jnp.float32),
            pltpu.VMEM((2, _CH), jnp.float32),
            pltpu.SemaphoreType.DMA,
        ],
        compiler_params=pltpu.CompilerParams(use_tc_tiling_on_sc=False),
    )
    return fn(boxes_p, s_p, order)


def _iou_rc(rx1, ry1, rx2, ry2, cx1, cy1, cx2, cy2):
    area_r = (rx2 - rx1) * (ry2 - ry1)
    area_c = (cx2 - cx1) * (cy2 - cy1)
    ltx = jnp.maximum(rx1, cx1)
    lty = jnp.maximum(ry1, cy1)
    rbx = jnp.minimum(rx2, cx2)
    rby = jnp.minimum(ry2, cy2)
    w = jnp.clip(rbx - ltx, 0.0, None)
    h = jnp.clip(rby - lty, 0.0, None)
    inter = w * h
    union = area_r + area_c - inter
    return inter / jnp.maximum(union, 1e-9)


def _nms_body(ct_ref, ss_ref, out_ref, keep_ref, cr_ref):
    f32 = jnp.float32

    for c in range(_NB):
        cr_ref[:, c * _B:(c + 1) * _B] = jnp.transpose(
            ct_ref[c * _B:(c + 1) * _B, :])

    keep_ref[:, :] = (ss_ref[:, :] > 0.0).astype(f32)

    riota = lax.broadcasted_iota(jnp.int32, (_B, _B), 0)
    ciota = lax.broadcasted_iota(jnp.int32, (_B, _B), 1)
    tri = (ciota > riota).astype(f32)

    def diag_cond(carry):
        d, count = carry
        return jnp.logical_and(d < _NB, count < _MAXD)

    def diag_body(carry):
        d, count = carry
        o = d * _B
        rx1 = ct_ref[pl.ds(o, _B), 0:1]
        ry1 = ct_ref[pl.ds(o, _B), 1:2]
        rx2 = ct_ref[pl.ds(o, _B), 2:3]
        ry2 = ct_ref[pl.ds(o, _B), 3:4]

        cx1 = cr_ref[0:1, pl.ds(o, _B)]
        cy1 = cr_ref[1:2, pl.ds(o, _B)]
        cx2 = cr_ref[2:3, pl.ds(o, _B)]
        cy2 = cr_ref[3:4, pl.ds(o, _B)]
        iou = _iou_rc(rx1, ry1, rx2, ry2, cx1, cy1, cx2, cy2)
        sf = jnp.where(iou > _T, tri, 0.0)

        k0 = keep_ref[pl.ds(d, 1), :]

        def fcond(c):
            _, changed, it = c
            return jnp.logical_and(changed, it <= _B)

        def fbody(c):
            k, _, it = c
            sup = jnp.dot(k, sf, preferred_element_type=f32)
            knew = jnp.where(sup > 0.5, 0.0, k0)
            return knew, jnp.any(knew != k), it + 1

        kf, _, _ = lax.while_loop(fcond, fbody, (k0, True, 0))
        keep_ref[pl.ds(d, 1), :] = kf
        count = count + jnp.sum(kf).astype(jnp.int32)

        def cbody(c, _):
            oc = c * _B
            ccx1 = cr_ref[0:1, pl.ds(oc, _B)]
            ccy1 = cr_ref[1:2, pl.ds(oc, _B)]
            ccx2 = cr_ref[2:3, pl.ds(oc, _B)]
            ccy2 = cr_ref[3:4, pl.ds(oc, _B)]
            iou_c = _iou_rc(rx1, ry1, rx2, ry2, ccx1, ccy1, ccx2, ccy2)
            sc = (iou_c > _T).astype(f32)
            sup = jnp.dot(kf, sc, preferred_element_type=f32)
            kc = keep_ref[pl.ds(c, 1), :]
            keep_ref[pl.ds(c, 1), :] = jnp.where(sup > 0.5, 0.0, kc)
            return 0

        lax.fori_loop(d + 1, _NB, cbody, 0)
        return d + 1, count

    _, count = lax.while_loop(diag_cond, diag_body, (jnp.int32(0), jnp.int32(0)))

    keep2 = keep_ref[:, :]
    jr = lax.broadcasted_iota(jnp.int32, (_NB, _B), 0)
    jc = lax.broadcasted_iota(jnp.int32, (_NB, _B), 1)
    jidx = jr * _B + jc
    lt_incl = (lax.broadcasted_iota(jnp.int32, (_B, _B), 0)
               <= lax.broadcasted_iota(jnp.int32, (_B, _B), 1)).astype(f32)
    intra = jnp.dot(keep2, lt_incl, preferred_element_type=f32)
    rows = jnp.sum(keep2, axis=1, keepdims=True)
    lt_strict = (lax.broadcasted_iota(jnp.int32, (_NB, _NB), 1)
                 < lax.broadcasted_iota(jnp.int32, (_NB, _NB), 0)).astype(f32)
    rowpfx = jnp.dot(lt_strict, rows, preferred_element_type=f32)
    c1 = intra + rowpfx
    cnt_f = count.astype(f32)
    pos = jnp.where(keep2 > 0.5, c1 - 1.0,
                    cnt_f + jidx.astype(f32) - c1)
    pos = jnp.minimum(pos, 127.0)
    piota = lax.broadcasted_iota(jnp.int32, (_B, 1), 0).astype(f32)
    acc = jnp.zeros((_B, 4), f32)
    sc_acc = jnp.zeros((_B, 1), f32)
    for c in range(_NB):
        m2 = (pos[c:c + 1, :] == piota).astype(f32)
        acc = acc + jnp.dot(m2, ct_ref[c * _B:(c + 1) * _B, :],
                            preferred_element_type=f32,
                            precision=lax.Precision.HIGHEST)
        sc_acc = sc_acc + jnp.sum(m2 * ss_ref[c:c + 1, :], axis=1,
                                  keepdims=True)
    out_ref[:, 0:4] = acc[0:_MAXD, :]
    out_ref[:, 4:5] = (sc_acc * (piota < cnt_f))[0:_MAXD, :]
    out_ref[:, 5:8] = jnp.zeros((_MAXD, 3), f32)


def _run_nms(ct4, ss2d, interpret=False):
    return pl.pallas_call(
        _nms_body,
        out_shape=jax.ShapeDtypeStruct((_MAXD, 8), jnp.float32),
        scratch_shapes=[
            pltpu.VMEM((_NB, _B), jnp.float32),
            pltpu.VMEM((4, _NP), jnp.float32),
        ],
        interpret=interpret,
    )(ct4, ss2d)


def kernel(boxes, scores):
    s = jnp.where(scores > 0.05, scores, -1.0)
    order = jnp.argsort(-s)
    pad = _NP - _N
    boxes_p = jnp.concatenate([boxes, jnp.zeros((pad, 4), jnp.float32)], axis=0)
    s_p = jnp.concatenate([s, jnp.full((pad,), -1.0, jnp.float32)], axis=0)
    order_p = jnp.concatenate(
        [order.astype(jnp.int32),
         jnp.arange(_N, _NP, dtype=jnp.int32)]).reshape(_NCH, _CH)
    ct4, ss = _sc_gather(boxes_p, s_p, order_p)
    out = _run_nms(ct4.reshape(_NP, 4), ss.reshape(_NB, _B))
    return out[:, :5]


# speedup vs baseline: 1.0272x; 1.0272x over previous
import functools

import jax
import jax.numpy as jnp
from jax import lax
from jax.experimental import pallas as pl
from jax.experimental.pallas import tpu as pltpu
from jax.experimental.pallas import tpu_sc as plsc

_N = 5000
_NP = 5120
_B = 128
_NB = _NP // _B
_T = 0.5
_MAXD = 100

_NW = 32
_RPW = _NP // _NW
_CH = 80
_NCH = _NP // _CH
_W = 16


def _sc_gather_body(table_hbm, order_hbm, out_hbm, idx_v, rows_v, sem):
    wid = lax.axis_index("s") * 2 + lax.axis_index("c")
    base = wid * (_RPW // _CH)
    pltpu.sync_copy(order_hbm.at[pl.ds(base, 2)], idx_v)
    c0 = pltpu.async_copy(table_hbm.at[idx_v.at[0]], rows_v.at[0], sem)
    c1 = pltpu.async_copy(table_hbm.at[idx_v.at[1]], rows_v.at[1], sem)
    c0.wait()
    c1.wait()
    pltpu.sync_copy(rows_v, out_hbm.at[pl.ds(base, 2)])


def _sc_gather(table, order):
    fn = pl.kernel(
        _sc_gather_body,
        out_type=jax.ShapeDtypeStruct((_NCH, _CH, _W), jnp.float32),
        mesh=plsc.VectorSubcoreMesh(core_axis_name="c", subcore_axis_name="s"),
        scratch_types=[
            pltpu.VMEM((2, _CH), jnp.int32),
            pltpu.VMEM((2, _CH, _W), jnp.float32),
            pltpu.SemaphoreType.DMA,
        ],
        compiler_params=pltpu.CompilerParams(use_tc_tiling_on_sc=False),
    )
    return fn(table, order)


def _iou_rc(rx1, ry1, rx2, ry2, cx1, cy1, cx2, cy2):
    area_r = (rx2 - rx1) * (ry2 - ry1)
    area_c = (cx2 - cx1) * (cy2 - cy1)
    ltx = jnp.maximum(rx1, cx1)
    lty = jnp.maximum(ry1, cy1)
    rbx = jnp.minimum(rx2, cx2)
    rby = jnp.minimum(ry2, cy2)
    w = jnp.clip(rbx - ltx, 0.0, None)
    h = jnp.clip(rby - lty, 0.0, None)
    inter = w * h
    union = area_r + area_c - inter
    return inter / jnp.maximum(union, 1e-9)


def _nms_body(ct_ref, out_ref, keep_ref, cr_ref, ss_ref):
    f32 = jnp.float32

    for c in range(_NB):
        t = jnp.transpose(ct_ref[c * _B:(c + 1) * _B, 0:8])
        cr_ref[:, c * _B:(c + 1) * _B] = t
        ss_ref[c:c + 1, :] = t[4:5, :]

    keep_ref[:, :] = (ss_ref[:, :] > 0.0).astype(f32)

    riota = lax.broadcasted_iota(jnp.int32, (_B, _B), 0)
    ciota = lax.broadcasted_iota(jnp.int32, (_B, _B), 1)
    tri = (ciota > riota).astype(f32)

    def diag_cond(carry):
        d, count = carry
        return jnp.logical_and(d < _NB, count < _MAXD)

    def diag_body(carry):
        d, count = carry
        o = d * _B
        rx1 = ct_ref[pl.ds(o, _B), 0:1]
        ry1 = ct_ref[pl.ds(o, _B), 1:2]
        rx2 = ct_ref[pl.ds(o, _B), 2:3]
        ry2 = ct_ref[pl.ds(o, _B), 3:4]

        cx1 = cr_ref[0:1, pl.ds(o, _B)]
        cy1 = cr_ref[1:2, pl.ds(o, _B)]
        cx2 = cr_ref[2:3, pl.ds(o, _B)]
        cy2 = cr_ref[3:4, pl.ds(o, _B)]
        iou = _iou_rc(rx1, ry1, rx2, ry2, cx1, cy1, cx2, cy2)
        sf = jnp.where(iou > _T, tri, 0.0)

        k0 = keep_ref[pl.ds(d, 1), :]

        def fcond(c):
            _, changed, it = c
            return jnp.logical_and(changed, it <= _B)

        def fbody(c):
            k, _, it = c
            sup = jnp.dot(k, sf, preferred_element_type=f32)
            knew = jnp.where(sup > 0.5, 0.0, k0)
            return knew, jnp.any(knew != k), it + 1

        kf, _, _ = lax.while_loop(fcond, fbody, (k0, True, 0))
        keep_ref[pl.ds(d, 1), :] = kf
        count = count + jnp.sum(kf).astype(jnp.int32)

        def cbody(c, _):
            oc = c * _B
            ccx1 = cr_ref[0:1, pl.ds(oc, _B)]
            ccy1 = cr_ref[1:2, pl.ds(oc, _B)]
            ccx2 = cr_ref[2:3, pl.ds(oc, _B)]
            ccy2 = cr_ref[3:4, pl.ds(oc, _B)]
            iou_c = _iou_rc(rx1, ry1, rx2, ry2, ccx1, ccy1, ccx2, ccy2)
            sc = (iou_c > _T).astype(f32)
            sup = jnp.dot(kf, sc, preferred_element_type=f32)
            kc = keep_ref[pl.ds(c, 1), :]
            keep_ref[pl.ds(c, 1), :] = jnp.where(sup > 0.5, 0.0, kc)
            return 0

        lax.fori_loop(d + 1, _NB, cbody, 0)
        return d + 1, count

    _, count = lax.while_loop(diag_cond, diag_body, (jnp.int32(0), jnp.int32(0)))

    keep2 = keep_ref[:, :]
    jr = lax.broadcasted_iota(jnp.int32, (_NB, _B), 0)
    jc = lax.broadcasted_iota(jnp.int32, (_NB, _B), 1)
    jidx = jr * _B + jc
    lt_incl = (lax.broadcasted_iota(jnp.int32, (_B, _B), 0)
               <= lax.broadcasted_iota(jnp.int32, (_B, _B), 1)).astype(f32)
    intra = jnp.dot(keep2, lt_incl, preferred_element_type=f32)
    rows = jnp.sum(keep2, axis=1, keepdims=True)
    lt_strict = (lax.broadcasted_iota(jnp.int32, (_NB, _NB), 1)
                 < lax.broadcasted_iota(jnp.int32, (_NB, _NB), 0)).astype(f32)
    rowpfx = jnp.dot(lt_strict, rows, preferred_element_type=f32)
    c1 = intra + rowpfx
    cnt_f = count.astype(f32)
    pos = jnp.where(keep2 > 0.5, c1 - 1.0,
                    cnt_f + jidx.astype(f32) - c1)
    pos = jnp.minimum(pos, 127.0)
    piota = lax.broadcasted_iota(jnp.int32, (_B, 1), 0).astype(f32)
    acc = jnp.zeros((_B, 8), f32)
    for c in range(_NB):
        m2 = (pos[c:c + 1, :] == piota).astype(f32)
        acc = acc + jnp.dot(m2, ct_ref[c * _B:(c + 1) * _B, 0:8],
                            preferred_element_type=f32,
                            precision=lax.Precision.HIGHEST)
    out_ref[:, 0:4] = acc[0:_MAXD, 0:4]
    out_ref[:, 4:5] = (acc[:, 4:5] * (piota < cnt_f))[0:_MAXD, :]
    out_ref[:, 5:8] = acc[0:_MAXD, 5:8]


def _run_nms(ct8, interpret=False):
    return pl.pallas_call(
        _nms_body,
        out_shape=jax.ShapeDtypeStruct((_MAXD, 8), jnp.float32),
        scratch_shapes=[
            pltpu.VMEM((_NB, _B), jnp.float32),
            pltpu.VMEM((8, _NP), jnp.float32),
            pltpu.VMEM((_NB, _B), jnp.float32),
        ],
        interpret=interpret,
    )(ct8)


def kernel(boxes, scores):
    s = jnp.where(scores > 0.05, scores, -1.0)
    order = jnp.argsort(-s)
    pad = _NP - _N
    col03 = jnp.concatenate([boxes, jnp.zeros((pad, 4), jnp.float32)], axis=0)
    col4 = jnp.concatenate([s, jnp.full((pad,), -1.0, jnp.float32)], axis=0)
    table = jnp.concatenate(
        [col03, col4[:, None], jnp.zeros((_NP, _W - 5), jnp.float32)], axis=1)
    order_p = jnp.concatenate(
        [order.astype(jnp.int32),
         jnp.arange(_N, _NP, dtype=jnp.int32)]).reshape(_NCH, _CH)
    ct8 = _sc_gather(table, order_p).reshape(_NP, _W)
    out = _run_nms(ct8)
    return out[:, :5]


# speedup vs baseline: 1.9983x; 1.9453x over previous
import functools

import jax
import jax.numpy as jnp
from jax import lax
from jax.experimental import pallas as pl
from jax.experimental.pallas import tpu as pltpu
from jax.experimental.pallas import tpu_sc as plsc

_N = 5000
_NP = 5120
_B = 128
_NB = _NP // _B
_T = 0.5
_MAXD = 100

_NW = 32
_RPW = _NP // _NW
_CH = 80
_NCH = _NP // _CH
_W = 16


def _sc_gather_body(table_hbm, order_hbm, out_hbm, idx_v, rows_v, sem):
    wid = lax.axis_index("s") * 2 + lax.axis_index("c")
    base = wid * (_RPW // _CH)
    pltpu.sync_copy(order_hbm.at[pl.ds(base, 2)], idx_v)
    c0 = pltpu.async_copy(table_hbm.at[idx_v.at[0]], rows_v.at[0], sem)
    c1 = pltpu.async_copy(table_hbm.at[idx_v.at[1]], rows_v.at[1], sem)
    c0.wait()
    c1.wait()
    pltpu.sync_copy(rows_v, out_hbm.at[pl.ds(base, 2)])


def _sc_gather(table, order):
    fn = pl.kernel(
        _sc_gather_body,
        out_type=jax.ShapeDtypeStruct((_NCH, _CH, _W), jnp.float32),
        mesh=plsc.VectorSubcoreMesh(core_axis_name="c", subcore_axis_name="s"),
        scratch_types=[
            pltpu.VMEM((2, _CH), jnp.int32),
            pltpu.VMEM((2, _CH, _W), jnp.float32),
            pltpu.SemaphoreType.DMA,
        ],
        compiler_params=pltpu.CompilerParams(use_tc_tiling_on_sc=False),
    )
    return fn(table, order)


def _iou_rc(rx1, ry1, rx2, ry2, cx1, cy1, cx2, cy2):
    area_r = (rx2 - rx1) * (ry2 - ry1)
    area_c = (cx2 - cx1) * (cy2 - cy1)
    ltx = jnp.maximum(rx1, cx1)
    lty = jnp.maximum(ry1, cy1)
    rbx = jnp.minimum(rx2, cx2)
    rby = jnp.minimum(ry2, cy2)
    w = jnp.clip(rbx - ltx, 0.0, None)
    h = jnp.clip(rby - lty, 0.0, None)
    inter = w * h
    union = area_r + area_c - inter
    return inter / jnp.maximum(union, 1e-9)


def _nms_body(cr_ref, ss_ref, out_ref, keep_ref, ct_ref):
    f32 = jnp.float32

    for c in range(_NB):
        ct_ref[c * _B:(c + 1) * _B, :] = jnp.transpose(
            cr_ref[:, c * _B:(c + 1) * _B])

    keep_ref[:, :] = (ss_ref[:, :] > 0.0).astype(f32)

    riota = lax.broadcasted_iota(jnp.int32, (_B, _B), 0)
    ciota = lax.broadcasted_iota(jnp.int32, (_B, _B), 1)
    tri = (ciota > riota).astype(f32)

    def diag_cond(carry):
        d, count = carry
        return jnp.logical_and(d < _NB, count < _MAXD)

    def diag_body(carry):
        d, count = carry
        o = d * _B
        rx1 = ct_ref[pl.ds(o, _B), 0:1]
        ry1 = ct_ref[pl.ds(o, _B), 1:2]
        rx2 = ct_ref[pl.ds(o, _B), 2:3]
        ry2 = ct_ref[pl.ds(o, _B), 3:4]

        cx1 = cr_ref[0:1, pl.ds(o, _B)]
        cy1 = cr_ref[1:2, pl.ds(o, _B)]
        cx2 = cr_ref[2:3, pl.ds(o, _B)]
        cy2 = cr_ref[3:4, pl.ds(o, _B)]
        iou = _iou_rc(rx1, ry1, rx2, ry2, cx1, cy1, cx2, cy2)
        sf = jnp.where(iou > _T, tri, 0.0)

        k0 = keep_ref[pl.ds(d, 1), :]

        def fcond(c):
            _, changed, it = c
            return jnp.logical_and(changed, it <= _B)

        def fbody(c):
            k, _, it = c
            sup = jnp.dot(k, sf, preferred_element_type=f32)
            knew = jnp.where(sup > 0.5, 0.0, k0)
            return knew, jnp.any(knew != k), it + 1

        kf, _, _ = lax.while_loop(fcond, fbody, (k0, True, 0))
        keep_ref[pl.ds(d, 1), :] = kf
        count = count + jnp.sum(kf).astype(jnp.int32)

        def cbody(c, _):
            oc = c * _B
            ccx1 = cr_ref[0:1, pl.ds(oc, _B)]
            ccy1 = cr_ref[1:2, pl.ds(oc, _B)]
            ccx2 = cr_ref[2:3, pl.ds(oc, _B)]
            ccy2 = cr_ref[3:4, pl.ds(oc, _B)]
            iou_c = _iou_rc(rx1, ry1, rx2, ry2, ccx1, ccy1, ccx2, ccy2)
            sc = (iou_c > _T).astype(f32)
            sup = jnp.dot(kf, sc, preferred_element_type=f32)
            kc = keep_ref[pl.ds(c, 1), :]
            keep_ref[pl.ds(c, 1), :] = jnp.where(sup > 0.5, 0.0, kc)
            return 0

        lax.fori_loop(d + 1, _NB, cbody, 0)
        return d + 1, count

    _, count = lax.while_loop(diag_cond, diag_body, (jnp.int32(0), jnp.int32(0)))

    keep2 = keep_ref[:, :]
    jr = lax.broadcasted_iota(jnp.int32, (_NB, _B), 0)
    jc = lax.broadcasted_iota(jnp.int32, (_NB, _B), 1)
    jidx = jr * _B + jc
    lt_incl = (lax.broadcasted_iota(jnp.int32, (_B, _B), 0)
               <= lax.broadcasted_iota(jnp.int32, (_B, _B), 1)).astype(f32)
    intra = jnp.dot(keep2, lt_incl, preferred_element_type=f32)
    rows = jnp.sum(keep2, axis=1, keepdims=True)
    lt_strict = (lax.broadcasted_iota(jnp.int32, (_NB, _NB), 1)
                 < lax.broadcasted_iota(jnp.int32, (_NB, _NB), 0)).astype(f32)
    rowpfx = jnp.dot(lt_strict, rows, preferred_element_type=f32)
    c1 = intra + rowpfx
    cnt_f = count.astype(f32)
    pos = jnp.where(keep2 > 0.5, c1 - 1.0,
                    cnt_f + jidx.astype(f32) - c1)
    pos = jnp.minimum(pos, 127.0)
    piota = lax.broadcasted_iota(jnp.int32, (_B, 1), 0).astype(f32)
    acc = jnp.zeros((_B, 8), f32)
    for c in range(_NB):
        m2 = (pos[c:c + 1, :] == piota).astype(f32)
        acc = acc + jnp.dot(m2, ct_ref[c * _B:(c + 1) * _B, 0:8],
                            preferred_element_type=f32,
                            precision=lax.Precision.HIGHEST)
    out_ref[:, 0:4] = acc[0:_MAXD, 0:4]
    out_ref[:, 4:5] = (acc[:, 4:5] * (piota < cnt_f))[0:_MAXD, :]
    out_ref[:, 5:8] = acc[0:_MAXD, 5:8]


def _run_nms(cr8, ss2d, interpret=False):
    return pl.pallas_call(
        _nms_body,
        out_shape=jax.ShapeDtypeStruct((_MAXD, 8), jnp.float32),
        scratch_shapes=[
            pltpu.VMEM((_NB, _B), jnp.float32),
            pltpu.VMEM((_NP, 8), jnp.float32),
        ],
        interpret=interpret,
    )(cr8, ss2d)


def kernel(boxes, scores):
    s = jnp.where(scores > 0.05, scores, -1.0)
    pad = _NP - _N
    s_p = jnp.concatenate([s, jnp.full((pad,), -1.0, jnp.float32)])
    b_p = jnp.concatenate([boxes, jnp.zeros((pad, 4), jnp.float32)], axis=0)
    srt = lax.sort((-s_p, b_p[:, 0], b_p[:, 1], b_p[:, 2], b_p[:, 3], s_p),
                   num_keys=1, is_stable=True)
    z = jnp.zeros(_NP, jnp.float32)
    cr8 = jnp.stack([srt[1], srt[2], srt[3], srt[4], srt[5], z, z, z])
    out = _run_nms(cr8, srt[5].reshape(_NB, _B))
    return out[:, :5]
